# Initial kernel scaffold; baseline (speedup 1.0000x reference)
import jax
import jax.numpy as jnp
from jax.experimental import pallas as pl


def kernel(x, edge_index, batch, clinical, Wd0, bd0, Wd1, bd1, Wd2, bd2, Wd3, bd3, Wd4, bd4, p1, p2, p3, p4, Wu0, bu0, Wu1, bu1, Wu2, bu2, Wu3, bu3):
    raise NotImplementedError("write your pallas kernel here")


# speedup vs baseline: 1.2661x; 1.2661x over previous
import functools

import jax
import jax.numpy as jnp
from jax import lax
from jax.experimental import pallas as pl
from jax.experimental.pallas import tpu as pltpu
from jax.experimental.pallas import tpu_sc as plsc

NN = 2048
EE = 32768
HH = 128
_HI = lax.Precision.HIGHEST


def _dot(a, b, dims):
    return lax.dot_general(a, b, (dims, ((), ())),
                           preferred_element_type=jnp.float32, precision=_HI)


def _mm(a, b):
    return _dot(a, b, ((1,), (0,)))


def _sc_build_body(edge_ref, out_ref, dstv, srcv, idxv, valv, zrow, shared, sem):
    c = lax.axis_index("c")
    s = lax.axis_index("s")
    epw = EE // 16
    base = s * epw
    pltpu.sync_copy(edge_ref.at[1, pl.ds(base, epw)], dstv)
    pltpu.sync_copy(edge_ref.at[0, pl.ds(base, epw)], srcv)
    rb = c * (NN // 2)
    lane = lax.iota(jnp.int32, 16)

    def zfill(j, _):
        zrow[pl.ds(j * 16, 16)] = jnp.zeros((16,), jnp.float32)
        return 0
    lax.fori_loop(0, 16384 // 16, zfill, 0)

    for q in range(2):
        cb = q * (NN // 2)

        zcs = [pltpu.async_copy(zrow,
                                shared.at[pl.ds(s * 65536 + t * 16384, 16384)],
                                sem) for t in range(4)]
        for zc in zcs:
            zc.wait()

        def body(j, _):
            d = dstv[pl.ds(j * 16, 16)]
            sr = srcv[pl.ds(j * 16, 16)]
            ok = (d >= rb) & (d < rb + 1024) & (sr >= cb) & (sr < cb + 1024)
            lin = (d - rb) * 1024 + (sr - cb)
            dump = j * 16 + lane
            idxv[pl.ds(j * 16, 16)] = jnp.where(ok, lin, dump)
            valv[pl.ds(j * 16, 16)] = jnp.where(ok, jnp.float32(1.0),
                                                jnp.float32(0.0))
            return 0
        lax.fori_loop(0, epw // 16, body, 0)

        plsc.subcore_barrier()
        pltpu.sync_copy(valv, shared.at[idxv], add=True)
        plsc.subcore_barrier()

        dcs = []
        for t in range(64):
            r = s * 64 + t
            dcs.append(pltpu.async_copy(
                shared.at[pl.ds(r * 1024, 1024)],
                out_ref.at[pl.ds((rb + r) * NN + cb, 1024)], sem))
        for dc in dcs:
            dc.wait()
        plsc.subcore_barrier()


def _sc_build(edge_index):
    mesh = plsc.VectorSubcoreMesh(core_axis_name="c", subcore_axis_name="s")
    fn = pl.kernel(
        _sc_build_body,
        out_type=jax.ShapeDtypeStruct((NN * NN,), jnp.float32),
        mesh=mesh,
        scratch_types=[
            pltpu.VMEM((EE // 16,), jnp.int32),
            pltpu.VMEM((EE // 16,), jnp.int32),
            pltpu.VMEM((EE // 16,), jnp.int32),
            pltpu.VMEM((EE // 16,), jnp.float32),
            pltpu.VMEM((16384,), jnp.float32),
            pltpu.VMEM_SHARED((1024 * 1024,), jnp.float32),
            pltpu.SemaphoreType.DMA,
        ],
    )
    return fn(edge_index).reshape(NN, NN)


def _sc_gather_body(k, n, a_ref, at_ref, perm_ref, r_ref, s_ref, idxv, rowsv, sem):
    c = lax.axis_index("c")
    s = lax.axis_index("s")
    wid = s * 2 + c
    nw = min(32, k // 8)
    rpw = k // nw

    @pl.when(wid < nw)
    def _():
        base = wid * rpw
        pltpu.sync_copy(perm_ref.at[pl.ds(base, rpw)], idxv)
        pltpu.async_copy(a_ref.at[idxv], rowsv, sem).wait()
        pltpu.sync_copy(rowsv, r_ref.at[pl.ds(base, rpw)])
        pltpu.async_copy(at_ref.at[idxv], rowsv, sem).wait()
        pltpu.sync_copy(rowsv, s_ref.at[pl.ds(base, rpw)])


def _sc_gather(a, at, perm, k, n):
    mesh = plsc.VectorSubcoreMesh(core_axis_name="c", subcore_axis_name="s")
    rpw = k // min(32, k // 8)
    fn = pl.kernel(
        functools.partial(_sc_gather_body, k, n),
        out_type=(jax.ShapeDtypeStruct((k, n), jnp.float32),
                  jax.ShapeDtypeStruct((k, n), jnp.float32)),
        mesh=mesh,
        scratch_types=[
            pltpu.VMEM((rpw,), jnp.int32),
            pltpu.VMEM((rpw, n), jnp.float32),
            pltpu.SemaphoreType.DMA,
        ],
    )
    return fn(a, at, perm)


def _diag_col(a_ref, n):
    ch = min(n, 256)
    eye = (lax.broadcasted_iota(jnp.int32, (ch, ch), 0) ==
           lax.broadcasted_iota(jnp.int32, (ch, ch), 1))
    parts = []
    for i in range(n // ch):
        blk = a_ref[i * ch:(i + 1) * ch, i * ch:(i + 1) * ch]
        parts.append(jnp.sum(jnp.where(eye, blk, 0.0), axis=1, keepdims=True))
    return jnp.concatenate(parts, axis=0)


def _gcn_ref(x, a_ref, w, b, f_col, n):
    ch = min(n, 256)
    nb = n // ch
    deg = jnp.concatenate(
        [jnp.sum(a_ref[i * ch:(i + 1) * ch, :], axis=1, keepdims=True)
         for i in range(nb)], axis=0) + f_col
    dinv = jnp.where(deg > 0.0, lax.rsqrt(deg), 0.0)
    z = _mm(x, w)
    u = dinv * z
    au = jnp.concatenate(
        [_mm(a_ref[i * ch:(i + 1) * ch, :], u) for i in range(nb)], axis=0)
    return dinv * (au + f_col * u) + b


def _gcn(x, a, w, b, f_col):
    deg = jnp.sum(a, axis=1, keepdims=True) + f_col
    dinv = jnp.where(deg > 0.0, lax.rsqrt(deg), 0.0)
    z = _mm(x, w)
    u = dinv * z
    return dinv * (_mm(a, u) + f_col * u) + b


def _rank_col(s_col, s_row, n):
    ii = lax.broadcasted_iota(jnp.int32, (n, 1), 0)
    ch = 256
    acc = jnp.zeros((n, 1), jnp.float32)
    for cidx in range(n // ch):
        sj = s_row[:, cidx * ch:(cidx + 1) * ch]
        jj = lax.broadcasted_iota(jnp.int32, (1, ch), 1) + cidx * ch
        gt = (sj > s_col).astype(jnp.float32)
        eq = ((sj == s_col) & (jj < ii)).astype(jnp.float32)
        acc = acc + jnp.sum(gt + eq, axis=1, keepdims=True)
    return acc


def _perm_from_scores(h, p_col, n, k):
    nrm = jnp.sqrt(jnp.sum(p_col * p_col))
    s_col = jnp.tanh(_mm(h, p_col) / nrm)
    s_row = jnp.swapaxes(s_col, 0, 1)
    rank_row = jnp.swapaxes(_rank_col(s_col, s_row, n), 0, 1)
    rank_i = rank_row.astype(jnp.int32)
    iota_k = lax.broadcasted_iota(jnp.int32, (k, 1), 0)
    ch = 256
    perm_col = jnp.zeros((k, 1), jnp.float32)
    for c in range(n // ch):
        ohc = (rank_i[:, c * ch:(c + 1) * ch] == iota_k).astype(jnp.float32)
        ic = (lax.broadcasted_iota(jnp.int32, (ch, 1), 0) + c * ch
              ).astype(jnp.float32)
        perm_col = perm_col + _mm(ohc, ic)
    return perm_col, s_col


def _onehot(perm_col_i32, k, n):
    return (perm_col_i32 == lax.broadcasted_iota(jnp.int32, (k, n), 1)
            ).astype(jnp.float32)


def _augment_pool(r_ref, s_ref, oh, k, n):
    ch = min(k, 256)
    nb = k // ch
    parts = []
    for i in range(nb):
        rp_c = r_ref[i * ch:(i + 1) * ch, :] + oh[i * ch:(i + 1) * ch, :]
        row = []
        for j in range(nb):
            sp_c = s_ref[j * ch:(j + 1) * ch, :] + oh[j * ch:(j + 1) * ch, :]
            row.append(_dot(rp_c, sp_c, ((1,), (1,))))
        blk = jnp.concatenate(row, axis=1)
        rr = lax.broadcasted_iota(jnp.int32, (ch, k), 0) + i * ch
        cc = lax.broadcasted_iota(jnp.int32, (ch, k), 1)
        parts.append(jnp.where(rr == cc, 0.0, blk))
    return jnp.concatenate(parts, axis=0)


def _tc_down0_body(x_ref, a_ref, w_ref, b_ref, p_ref,
                   h_ref, perm_ref, f_ref):
    diag = _diag_col(a_ref, NN)
    f_col = jnp.where(diag == 0.0, 2.0, 0.0)
    h = jax.nn.relu(_gcn_ref(x_ref[...], a_ref, w_ref[...], b_ref[...],
                             f_col, NN))
    h_ref[...] = h
    perm_col, _ = _perm_from_scores(h, p_ref[...], NN, NN // 2)
    perm_ref[...] = perm_col.astype(jnp.int32)
    f_ref[...] = f_col


def _tc_xpose_body(a_ref, at_ref):
    at_ref[...] = jnp.swapaxes(a_ref[...], 0, 1)


def _tc_xpose(a, m):
    return pl.pallas_call(
        _tc_xpose_body,
        out_shape=jax.ShapeDtypeStruct((m, m), jnp.float32),
    )(a)


def _tc_pool_body(n, k, h_ref, r_ref, s_ref, perm_ref, pi_ref,
                  a_ref_o, xp_ref):
    h_prev = h_ref[...]
    oh = _onehot(perm_ref[...], k, n)
    nrm = jnp.sqrt(jnp.sum(pi_ref[...] * pi_ref[...]))
    s_col = jnp.tanh(_mm(h_prev, pi_ref[...]) / nrm)
    sperm = _mm(oh, s_col)
    xp_ref[...] = _mm(oh, h_prev) * sperm
    a_ref_o[...] = _augment_pool(r_ref, s_ref, oh, k, n)


def _tc_gcn_body(k, last, xp_ref, a_ref, w_ref, b_ref, pn_ref, *out_refs):
    two = jnp.full((k, 1), 2.0, jnp.float32)
    hn = jax.nn.relu(_gcn_ref(xp_ref[...], a_ref, w_ref[...], b_ref[...],
                              two, k))
    out_refs[0][...] = hn
    if not last:
        perm2_col, _ = _perm_from_scores(hn, pn_ref[...], k, k // 2)
        out_refs[1][...] = perm2_col.astype(jnp.int32)


def _tc_up_body(h4_ref, h3_ref, h2_ref, h1_ref, h0_ref,
                a3_ref, a2_ref, a1_ref, a0_ref, f0_ref,
                p4_ref, p3_ref, p2_ref, p1_ref,
                wu0_ref, bu0_ref, wu1_ref, bu1_ref,
                wu2_ref, bu2_ref, wu3_ref, bu3_ref,
                out_ref):
    x = h4_ref[...]
    sizes = ((128, 256), (256, 512), (512, 1024), (1024, 2048))
    perms = (p4_ref, p3_ref, p2_ref, p1_ref)
    res = (h3_ref, h2_ref, h1_ref, h0_ref)
    adjs = (a3_ref, a2_ref, a1_ref, a0_ref)
    wus = ((wu0_ref, bu0_ref), (wu1_ref, bu1_ref),
           (wu2_ref, bu2_ref), (wu3_ref, bu3_ref))
    for i in range(4):
        k, n = sizes[i]
        oh = _onehot(perms[i][...], k, n)
        up = _dot(oh, x, ((0,), (0,)))
        x = res[i][...] + up
        if i < 3:
            f_col = jnp.full((n, 1), 2.0, jnp.float32)
        else:
            f_col = f0_ref[...]
        w, b = wus[i]
        x = _gcn_ref(x, adjs[i], w[...], b[...], f_col, n)
        if i < 3:
            x = jax.nn.relu(x)
    out_ref[...] = jnp.sum(x, axis=0, keepdims=True) / x.shape[0]


def _tc_down0(x, a, w, b, p):
    return pl.pallas_call(
        _tc_down0_body,
        out_shape=[
            jax.ShapeDtypeStruct((NN, HH), jnp.float32),
            jax.ShapeDtypeStruct((NN // 2, 1), jnp.int32),
            jax.ShapeDtypeStruct((NN, 1), jnp.float32),
        ],
    )(x, a, w, b, p)


def _tc_pool(h, r, s, perm, pi, n, k):
    return pl.pallas_call(
        functools.partial(_tc_pool_body, n, k),
        out_shape=[
            jax.ShapeDtypeStruct((k, k), jnp.float32),
            jax.ShapeDtypeStruct((k, HH), jnp.float32),
        ],
    )(h, r, s, perm, pi)


def _tc_gcn(xp, a, w, b, pn, k, last):
    outs = [jax.ShapeDtypeStruct((k, HH), jnp.float32)]
    if not last:
        outs.append(jax.ShapeDtypeStruct((k // 2, 1), jnp.int32))
    return pl.pallas_call(
        functools.partial(_tc_gcn_body, k, last),
        out_shape=outs,
    )(xp, a, w, b, pn)


def _tc_up(h4, h3, h2, h1, h0, a3, a2, a1, a0, f0, p4, p3, p2, p1, wus):
    args = [h4, h3, h2, h1, h0, a3, a2, a1, a0, f0, p4, p3, p2, p1]
    for w, b in wus:
        args += [w, b]
    return pl.pallas_call(
        _tc_up_body,
        out_shape=jax.ShapeDtypeStruct((1, HH), jnp.float32),
    )(*args)


def kernel(x, edge_index, batch, clinical, Wd0, bd0, Wd1, bd1, Wd2, bd2,
           Wd3, bd3, Wd4, bd4, p1, p2, p3, p4, Wu0, bu0, Wu1, bu1, Wu2, bu2,
           Wu3, bu3):
    bds = [b.reshape(1, HH) for b in (bd0, bd1, bd2, bd3, bd4)]
    bus = [b.reshape(1, HH) for b in (bu0, bu1, bu2, bu3)]
    ps = [p.reshape(HH, 1) for p in (p1, p2, p3, p4)]
    wds = [Wd0, Wd1, Wd2, Wd3, Wd4]
    wus = [Wu0, Wu1, Wu2, Wu3]

    a0 = _sc_build(edge_index)
    h0, perm1, f0 = _tc_down0(x, a0, wds[0], bds[0], ps[0])
    a0t = _tc_xpose(a0, NN)

    hs = [h0]
    adjs = [a0]
    permcols = [perm1]
    a_cur, at_cur = a0, a0t
    h_cur = h0
    ns = [2048, 1024, 512, 256]
    for i in range(4):
        n, k = ns[i], ns[i] // 2
        perm_flat = permcols[i].reshape(k)
        r, s = _sc_gather(a_cur, at_cur, perm_flat, k, n)
        last = i == 3
        pn = ps[i + 1] if not last else ps[i]
        a2, xp = _tc_pool(h_cur, r, s, permcols[i], ps[i], n, k)
        res = _tc_gcn(xp, a2, wds[i + 1], bds[i + 1], pn, k, last)
        if last:
            h_cur = res[0]
        else:
            h_cur, perm_next = res
            a_cur = a2
            at_cur = _tc_xpose(a2, k)
            adjs.append(a2)
            hs.append(h_cur)
            permcols.append(perm_next)

    out = _tc_up(h_cur, hs[3], hs[2], hs[1], hs[0],
                 adjs[3], adjs[2], adjs[1], adjs[0], f0,
                 permcols[3], permcols[2], permcols[1], permcols[0],
                 list(zip(wus, bus)))
    return out


# speedup vs baseline: 1.2752x; 1.0072x over previous
import functools

import jax
import jax.numpy as jnp
from jax import lax
from jax.experimental import pallas as pl
from jax.experimental.pallas import tpu as pltpu
from jax.experimental.pallas import tpu_sc as plsc

NN = 2048
EE = 32768
HH = 128
_HI = lax.Precision.HIGHEST


def _dot(a, b, dims):
    return lax.dot_general(a, b, (dims, ((), ())),
                           preferred_element_type=jnp.float32, precision=_HI)


def _mm(a, b):
    return _dot(a, b, ((1,), (0,)))


def _sc_build_body(edge_ref, out_ref, dstv, srcv, idxv, valv, zrow, shared, sem):
    c = lax.axis_index("c")
    s = lax.axis_index("s")
    epw = EE // 16
    base = s * epw
    pltpu.sync_copy(edge_ref.at[1, pl.ds(base, epw)], dstv)
    pltpu.sync_copy(edge_ref.at[0, pl.ds(base, epw)], srcv)
    rb = c * (NN // 2)
    lane = lax.iota(jnp.int32, 16)

    def zfill(j, _):
        zrow[pl.ds(j * 16, 16)] = jnp.zeros((16,), jnp.float32)
        return 0
    lax.fori_loop(0, 16384 // 16, zfill, 0)

    for q in range(2):
        cb = q * (NN // 2)

        zcs = [pltpu.async_copy(zrow,
                                shared.at[pl.ds(s * 65536 + t * 16384, 16384)],
                                sem) for t in range(4)]
        for zc in zcs:
            zc.wait()

        def body(j, _):
            d = dstv[pl.ds(j * 16, 16)]
            sr = srcv[pl.ds(j * 16, 16)]
            ok = (d >= rb) & (d < rb + 1024) & (sr >= cb) & (sr < cb + 1024)
            lin = (d - rb) * 1024 + (sr - cb)
            dump = j * 16 + lane
            idxv[pl.ds(j * 16, 16)] = jnp.where(ok, lin, dump)
            valv[pl.ds(j * 16, 16)] = jnp.where(ok, jnp.float32(1.0),
                                                jnp.float32(0.0))
            return 0
        lax.fori_loop(0, epw // 16, body, 0)

        plsc.subcore_barrier()
        pltpu.sync_copy(valv, shared.at[idxv], add=True)
        plsc.subcore_barrier()

        dcs = []
        for t in range(64):
            r = s * 64 + t
            dcs.append(pltpu.async_copy(
                shared.at[pl.ds(r * 1024, 1024)],
                out_ref.at[pl.ds((rb + r) * NN + cb, 1024)], sem))
        for dc in dcs:
            dc.wait()
        plsc.subcore_barrier()


def _sc_build(edge_index):
    mesh = plsc.VectorSubcoreMesh(core_axis_name="c", subcore_axis_name="s")
    fn = pl.kernel(
        _sc_build_body,
        out_type=jax.ShapeDtypeStruct((NN * NN,), jnp.float32),
        mesh=mesh,
        scratch_types=[
            pltpu.VMEM((EE // 16,), jnp.int32),
            pltpu.VMEM((EE // 16,), jnp.int32),
            pltpu.VMEM((EE // 16,), jnp.int32),
            pltpu.VMEM((EE // 16,), jnp.float32),
            pltpu.VMEM((16384,), jnp.float32),
            pltpu.VMEM_SHARED((1024 * 1024,), jnp.float32),
            pltpu.SemaphoreType.DMA,
        ],
    )
    return fn(edge_index).reshape(NN, NN)


def _sc_gather_body(k, n, a_ref, at_ref, perm_ref, r_ref, s_ref, idxv, rowsv, sem):
    c = lax.axis_index("c")
    s = lax.axis_index("s")
    wid = s * 2 + c
    nw = min(32, k // 8)
    rpw = k // nw

    @pl.when(wid < nw)
    def _():
        base = wid * rpw
        pltpu.sync_copy(perm_ref.at[pl.ds(base, rpw)], idxv)
        pltpu.async_copy(a_ref.at[idxv], rowsv, sem).wait()
        pltpu.sync_copy(rowsv, r_ref.at[pl.ds(base, rpw)])
        pltpu.async_copy(at_ref.at[idxv], rowsv, sem).wait()
        pltpu.sync_copy(rowsv, s_ref.at[pl.ds(base, rpw)])


def _sc_gather(a, at, perm, k, n):
    mesh = plsc.VectorSubcoreMesh(core_axis_name="c", subcore_axis_name="s")
    rpw = k // min(32, k // 8)
    fn = pl.kernel(
        functools.partial(_sc_gather_body, k, n),
        out_type=(jax.ShapeDtypeStruct((k, n), jnp.float32),
                  jax.ShapeDtypeStruct((k, n), jnp.float32)),
        mesh=mesh,
        scratch_types=[
            pltpu.VMEM((rpw,), jnp.int32),
            pltpu.VMEM((rpw, n), jnp.float32),
            pltpu.SemaphoreType.DMA,
        ],
    )
    return fn(a, at, perm)


def _diag_col(a_ref, n):
    ch = min(n, 256)
    eye = (lax.broadcasted_iota(jnp.int32, (ch, ch), 0) ==
           lax.broadcasted_iota(jnp.int32, (ch, ch), 1))
    parts = []
    for i in range(n // ch):
        blk = a_ref[i * ch:(i + 1) * ch, i * ch:(i + 1) * ch]
        parts.append(jnp.sum(jnp.where(eye, blk, 0.0), axis=1, keepdims=True))
    return jnp.concatenate(parts, axis=0)


def _gcn_ref(x, a_ref, w, b, f_col, n):
    ch = min(n, 256)
    nb = n // ch
    deg = jnp.concatenate(
        [jnp.sum(a_ref[i * ch:(i + 1) * ch, :], axis=1, keepdims=True)
         for i in range(nb)], axis=0) + f_col
    dinv = jnp.where(deg > 0.0, lax.rsqrt(deg), 0.0)
    z = _mm(x, w)
    u = dinv * z
    au = jnp.concatenate(
        [_mm(a_ref[i * ch:(i + 1) * ch, :], u) for i in range(nb)], axis=0)
    return dinv * (au + f_col * u) + b


def _gcn(x, a, w, b, f_col):
    deg = jnp.sum(a, axis=1, keepdims=True) + f_col
    dinv = jnp.where(deg > 0.0, lax.rsqrt(deg), 0.0)
    z = _mm(x, w)
    u = dinv * z
    return dinv * (_mm(a, u) + f_col * u) + b


def _rank_col(s_col, s_row, n):
    ii = lax.broadcasted_iota(jnp.int32, (n, 1), 0)
    ch = 256
    acc = jnp.zeros((n, 1), jnp.float32)
    for cidx in range(n // ch):
        sj = s_row[:, cidx * ch:(cidx + 1) * ch]
        jj = lax.broadcasted_iota(jnp.int32, (1, ch), 1) + cidx * ch
        gt = (sj > s_col).astype(jnp.float32)
        eq = ((sj == s_col) & (jj < ii)).astype(jnp.float32)
        acc = acc + jnp.sum(gt + eq, axis=1, keepdims=True)
    return acc


def _perm_from_scores(h, p_col, n, k):
    nrm = jnp.sqrt(jnp.sum(p_col * p_col))
    s_col = jnp.tanh(_mm(h, p_col) / nrm)
    s_row = jnp.swapaxes(s_col, 0, 1)
    rank_row = jnp.swapaxes(_rank_col(s_col, s_row, n), 0, 1)
    rank_i = rank_row.astype(jnp.int32)
    iota_k = lax.broadcasted_iota(jnp.int32, (k, 1), 0)
    ch = 256
    perm_col = jnp.zeros((k, 1), jnp.float32)
    for c in range(n // ch):
        ohc = (rank_i[:, c * ch:(c + 1) * ch] == iota_k).astype(jnp.float32)
        ic = (lax.broadcasted_iota(jnp.int32, (ch, 1), 0) + c * ch
              ).astype(jnp.float32)
        perm_col = perm_col + _mm(ohc, ic)
    return perm_col, s_col


def _onehot(perm_col_i32, k, n):
    return (perm_col_i32 == lax.broadcasted_iota(jnp.int32, (k, n), 1)
            ).astype(jnp.float32)


def _augment_pool(r_ref, s_ref, oh, k, n):
    ch = min(k, 256)
    nb = k // ch
    parts = []
    for i in range(nb):
        rp_c = r_ref[i * ch:(i + 1) * ch, :] + oh[i * ch:(i + 1) * ch, :]
        row = []
        for j in range(nb):
            sp_c = s_ref[j * ch:(j + 1) * ch, :] + oh[j * ch:(j + 1) * ch, :]
            row.append(_dot(rp_c, sp_c, ((1,), (1,))))
        blk = jnp.concatenate(row, axis=1)
        rr = lax.broadcasted_iota(jnp.int32, (ch, k), 0) + i * ch
        cc = lax.broadcasted_iota(jnp.int32, (ch, k), 1)
        parts.append(jnp.where(rr == cc, 0.0, blk))
    return jnp.concatenate(parts, axis=0)


def _tc_down0_body(x_ref, a_ref, w_ref, b_ref, p_ref,
                   h_ref, perm_ref, f_ref):
    diag = _diag_col(a_ref, NN)
    f_col = jnp.where(diag == 0.0, 2.0, 0.0)
    h = jax.nn.relu(_gcn_ref(x_ref[...], a_ref, w_ref[...], b_ref[...],
                             f_col, NN))
    h_ref[...] = h
    perm_col, _ = _perm_from_scores(h, p_ref[...], NN, NN // 2)
    perm_ref[...] = perm_col.astype(jnp.int32)
    f_ref[...] = f_col


def _tc_xpose_body(a_ref, at_ref):
    at_ref[...] = jnp.swapaxes(a_ref[...], 0, 1)


def _tc_xpose(a, m):
    return pl.pallas_call(
        _tc_xpose_body,
        out_shape=jax.ShapeDtypeStruct((m, m), jnp.float32),
    )(a)


def _tc_pool_body(n, k, h_ref, r_ref, s_ref, perm_ref, pi_ref,
                  a_ref_o, at_ref_o, xp_ref):
    h_prev = h_ref[...]
    oh = _onehot(perm_ref[...], k, n)
    nrm = jnp.sqrt(jnp.sum(pi_ref[...] * pi_ref[...]))
    s_col = jnp.tanh(_mm(h_prev, pi_ref[...]) / nrm)
    sperm = _mm(oh, s_col)
    xp_ref[...] = _mm(oh, h_prev) * sperm
    a2 = _augment_pool(r_ref, s_ref, oh, k, n)
    a_ref_o[...] = a2
    at_ref_o[...] = jnp.swapaxes(a2, 0, 1)


def _tc_fused_body(n, k, last, h_ref, r_ref, s_ref, perm_ref, pi_ref,
                   w_ref, b_ref, pn_ref, *out_refs):
    h_prev = h_ref[...]
    oh = _onehot(perm_ref[...], k, n)
    nrm = jnp.sqrt(jnp.sum(pi_ref[...] * pi_ref[...]))
    s_col = jnp.tanh(_mm(h_prev, pi_ref[...]) / nrm)
    sperm = _mm(oh, s_col)
    xp = _mm(oh, h_prev) * sperm
    a2 = _augment_pool(r_ref, s_ref, oh, k, n)
    two = jnp.full((k, 1), 2.0, jnp.float32)
    hn = jax.nn.relu(_gcn(xp, a2, w_ref[...], b_ref[...], two))
    if last:
        out_refs[0][...] = hn
    else:
        a_o, at_o, h_o, perm_o = out_refs
        a_o[...] = a2
        at_o[...] = jnp.swapaxes(a2, 0, 1)
        h_o[...] = hn
        perm2_col, _ = _perm_from_scores(hn, pn_ref[...], k, k // 2)
        perm_o[...] = perm2_col.astype(jnp.int32)


def _tc_gcn_body(k, last, xp_ref, a_ref, w_ref, b_ref, pn_ref, *out_refs):
    two = jnp.full((k, 1), 2.0, jnp.float32)
    hn = jax.nn.relu(_gcn_ref(xp_ref[...], a_ref, w_ref[...], b_ref[...],
                              two, k))
    out_refs[0][...] = hn
    if not last:
        perm2_col, _ = _perm_from_scores(hn, pn_ref[...], k, k // 2)
        out_refs[1][...] = perm2_col.astype(jnp.int32)


def _tc_up_body(h4_ref, h3_ref, h2_ref, h1_ref, h0_ref,
                a3_ref, a2_ref, a1_ref, a0_ref, f0_ref,
                p4_ref, p3_ref, p2_ref, p1_ref,
                wu0_ref, bu0_ref, wu1_ref, bu1_ref,
                wu2_ref, bu2_ref, wu3_ref, bu3_ref,
                out_ref):
    x = h4_ref[...]
    sizes = ((128, 256), (256, 512), (512, 1024), (1024, 2048))
    perms = (p4_ref, p3_ref, p2_ref, p1_ref)
    res = (h3_ref, h2_ref, h1_ref, h0_ref)
    adjs = (a3_ref, a2_ref, a1_ref, a0_ref)
    wus = ((wu0_ref, bu0_ref), (wu1_ref, bu1_ref),
           (wu2_ref, bu2_ref), (wu3_ref, bu3_ref))
    for i in range(4):
        k, n = sizes[i]
        oh = _onehot(perms[i][...], k, n)
        up = _dot(oh, x, ((0,), (0,)))
        x = res[i][...] + up
        if i < 3:
            f_col = jnp.full((n, 1), 2.0, jnp.float32)
        else:
            f_col = f0_ref[...]
        w, b = wus[i]
        x = _gcn_ref(x, adjs[i], w[...], b[...], f_col, n)
        if i < 3:
            x = jax.nn.relu(x)
    out_ref[...] = jnp.sum(x, axis=0, keepdims=True) / x.shape[0]


def _tc_down0(x, a, w, b, p):
    return pl.pallas_call(
        _tc_down0_body,
        out_shape=[
            jax.ShapeDtypeStruct((NN, HH), jnp.float32),
            jax.ShapeDtypeStruct((NN // 2, 1), jnp.int32),
            jax.ShapeDtypeStruct((NN, 1), jnp.float32),
        ],
    )(x, a, w, b, p)


def _tc_pool(h, r, s, perm, pi, n, k):
    return pl.pallas_call(
        functools.partial(_tc_pool_body, n, k),
        out_shape=[
            jax.ShapeDtypeStruct((k, k), jnp.float32),
            jax.ShapeDtypeStruct((k, k), jnp.float32),
            jax.ShapeDtypeStruct((k, HH), jnp.float32),
        ],
    )(h, r, s, perm, pi)


def _tc_fused(h, r, s, perm, pi, w, b, pn, n, k, last):
    if last:
        outs = [jax.ShapeDtypeStruct((k, HH), jnp.float32)]
    else:
        outs = [
            jax.ShapeDtypeStruct((k, k), jnp.float32),
            jax.ShapeDtypeStruct((k, k), jnp.float32),
            jax.ShapeDtypeStruct((k, HH), jnp.float32),
            jax.ShapeDtypeStruct((k // 2, 1), jnp.int32),
        ]
    return pl.pallas_call(
        functools.partial(_tc_fused_body, n, k, last),
        out_shape=outs,
    )(h, r, s, perm, pi, w, b, pn)


def _tc_gcn(xp, a, w, b, pn, k, last):
    outs = [jax.ShapeDtypeStruct((k, HH), jnp.float32)]
    if not last:
        outs.append(jax.ShapeDtypeStruct((k // 2, 1), jnp.int32))
    return pl.pallas_call(
        functools.partial(_tc_gcn_body, k, last),
        out_shape=outs,
    )(xp, a, w, b, pn)


def _tc_up(h4, h3, h2, h1, h0, a3, a2, a1, a0, f0, p4, p3, p2, p1, wus):
    args = [h4, h3, h2, h1, h0, a3, a2, a1, a0, f0, p4, p3, p2, p1]
    for w, b in wus:
        args += [w, b]
    return pl.pallas_call(
        _tc_up_body,
        out_shape=jax.ShapeDtypeStruct((1, HH), jnp.float32),
    )(*args)


def kernel(x, edge_index, batch, clinical, Wd0, bd0, Wd1, bd1, Wd2, bd2,
           Wd3, bd3, Wd4, bd4, p1, p2, p3, p4, Wu0, bu0, Wu1, bu1, Wu2, bu2,
           Wu3, bu3):
    bds = [b.reshape(1, HH) for b in (bd0, bd1, bd2, bd3, bd4)]
    bus = [b.reshape(1, HH) for b in (bu0, bu1, bu2, bu3)]
    ps = [p.reshape(HH, 1) for p in (p1, p2, p3, p4)]
    wds = [Wd0, Wd1, Wd2, Wd3, Wd4]
    wus = [Wu0, Wu1, Wu2, Wu3]

    a0 = _sc_build(edge_index)
    h0, perm1, f0 = _tc_down0(x, a0, wds[0], bds[0], ps[0])
    a0t = _tc_xpose(a0, NN)

    hs = [h0]
    adjs = [a0]
    permcols = [perm1]
    a_cur, at_cur = a0, a0t
    h_cur = h0
    ns = [2048, 1024, 512, 256]
    for i in range(4):
        n, k = ns[i], ns[i] // 2
        perm_flat = permcols[i].reshape(k)
        r, s = _sc_gather(a_cur, at_cur, perm_flat, k, n)
        last = i == 3
        pn = ps[i + 1] if not last else ps[i]
        if i == 0:
            a2, a2t, xp = _tc_pool(h_cur, r, s, permcols[i], ps[i], n, k)
            h_cur, perm_next = _tc_gcn(xp, a2, wds[1], bds[1], pn, k, False)
            a_cur, at_cur = a2, a2t
            adjs.append(a2)
            hs.append(h_cur)
            permcols.append(perm_next)
        else:
            res = _tc_fused(h_cur, r, s, permcols[i], ps[i], wds[i + 1],
                            bds[i + 1], pn, n, k, last)
            if last:
                h_cur = res[0]
            else:
                a_cur, at_cur, h_cur, perm_next = res
                adjs.append(a_cur)
                hs.append(h_cur)
                permcols.append(perm_next)

    out = _tc_up(h_cur, hs[3], hs[2], hs[1], hs[0],
                 adjs[3], adjs[2], adjs[1], adjs[0], f0,
                 permcols[3], permcols[2], permcols[1], permcols[0],
                 list(zip(wus, bus)))
    return out
